# baseline (speedup 1.0000x reference)
import functools

import jax
import jax.numpy as jnp
from jax import lax
from jax.experimental import pallas as pl
from jax.experimental.pallas import tpu as pltpu
from jax.experimental.pallas import tpu_sc as plsc

EMB = 32
LANES = 16
CHUNK = 128


@functools.lru_cache(maxsize=None)
def _build(batch):
    nc, ns = 2, 16
    nw = nc * ns
    per_w = batch // nw
    n_chunks = per_w // CHUNK
    n_groups = per_w // LANES
    mesh = plsc.VectorSubcoreMesh(core_axis_name="c", subcore_axis_name="s")

    @functools.partial(
        pl.kernel,
        mesh=mesh,
        compiler_params=pltpu.CompilerParams(
            needs_layout_passes=False, use_tc_tiling_on_sc=False),
        out_type=jax.ShapeDtypeStruct((nw, per_w), jnp.float32),
        scratch_types=[
            pltpu.VMEM((n_chunks, CHUNK), jnp.int32),
            pltpu.VMEM((n_chunks, CHUNK), jnp.int32),
            pltpu.VMEM((n_chunks, CHUNK, EMB), jnp.float32),
            pltpu.VMEM((n_chunks, CHUNK, EMB), jnp.float32),
            pltpu.VMEM((n_chunks, CHUNK), jnp.float32),
            pltpu.VMEM((n_chunks, CHUNK), jnp.float32),
            pltpu.VMEM((per_w,), jnp.float32),
            pltpu.SemaphoreType.DMA,
        ],
    )
    def k(uemb, memb, ubias, mbias, uids, mids, out_hbm,
          uid_v, mid_v, ur_v, mr_v, ub_v, mb_v, out_v, sem):
        wid = lax.axis_index("s") * nc + lax.axis_index("c")
        pltpu.sync_copy(uids.at[wid], uid_v)
        pltpu.sync_copy(mids.at[wid], mid_v)
        copies = []
        for c in range(n_chunks):
            copies.append(pltpu.async_copy(uemb.at[uid_v.at[c]], ur_v.at[c], sem))
            copies.append(pltpu.async_copy(memb.at[mid_v.at[c]], mr_v.at[c], sem))
            copies.append(pltpu.async_copy(ubias.at[uid_v.at[c]], ub_v.at[c], sem))
            copies.append(pltpu.async_copy(mbias.at[mid_v.at[c]], mb_v.at[c], sem))
        for cp in copies:
            cp.wait()

        lanes = lax.iota(jnp.int32, 16)
        groups_per_chunk = CHUNK // LANES

        def group(g, carry):
            ci = g // groups_per_chunk
            s0 = (g % groups_per_chunk) * LANES
            c = jnp.full((16,), ci, jnp.int32)
            rows = lanes + s0
            acc = ub_v[ci, pl.ds(s0, LANES)] + mb_v[ci, pl.ds(s0, LANES)]
            for d in range(EMB):
                ds_ = jnp.full((16,), d, jnp.int32)
                acc = acc + (plsc.load_gather(ur_v, [c, rows, ds_])
                             * plsc.load_gather(mr_v, [c, rows, ds_]))
            out_v[pl.ds(g * LANES, LANES)] = acc
            return carry

        lax.fori_loop(0, n_groups, group, 0)
        pltpu.sync_copy(out_v, out_hbm.at[wid])

    return k, nw, per_w, n_chunks


def kernel(user_ids, movie_ids, user_emb, movie_emb, user_bias, movie_bias):
    batch = user_ids.shape[0]
    k, nw, per_w, n_chunks = _build(batch)
    uids = user_ids.astype(jnp.int32).reshape(nw, n_chunks, CHUNK)
    mids = movie_ids.astype(jnp.int32).reshape(nw, n_chunks, CHUNK)
    out = k(user_emb, movie_emb,
            user_bias.reshape(-1), movie_bias.reshape(-1), uids, mids)
    return out.reshape(batch)
